# baseline (speedup 1.0000x reference)
import functools

import jax
import jax.numpy as jnp
from jax import lax
from jax.experimental import pallas as pl
from jax.experimental.pallas import tpu as pltpu
from jax.experimental.pallas import tpu_sc as plsc

N = 100000
E = 1600000
OBS = 8
SD = 16
ED = 16
DP = 16

NC = 2
NS = 16
NW = NC * NS
CH = 128
GRP = 1024
GROUPS = 25
SPLIT = 2
PER_TILE = GROUPS * GRP
CHUNK_E = NW * PER_TILE
EP = SPLIT * CHUNK_E
NP = 102400
NT = 102400
ZR = NT // NS

_sc_params = pltpu.CompilerParams(use_tc_tiling_on_sc=False)


@functools.lru_cache(maxsize=1)
def _sc_mesh():
    return plsc.VectorSubcoreMesh(core_axis_name="c", subcore_axis_name="s",
                                  num_cores=NC, num_subcores=NS)


_RB = 2048
_RBP = _RB // 8


def _kron_mlp(params):
    w0, b0, w1, b1, w2, b2 = params
    i8 = jnp.eye(8, dtype=jnp.float32)
    return (jnp.kron(i8, w0), jnp.tile(b0, 8)[None, :],
            jnp.kron(i8, w1), jnp.tile(b1, 8)[None, :],
            jnp.kron(i8, w2), jnp.tile(b2, 8)[None, :])


def _bd_mlp(x, kw):
    bd0, b0t, bd1, b1t, bd2, b2t = kw
    h = jax.nn.relu(jnp.dot(x, bd0[...], preferred_element_type=jnp.float32)
                    + b0t[...])
    h = jax.nn.relu(jnp.dot(h, bd1[...], preferred_element_type=jnp.float32)
                    + b1t[...])
    return jnp.dot(h, bd2[...], preferred_element_type=jnp.float32) + b2t[...]


def _percell_body(cells_ref,
                  c0, c1, c2, c3, c4, c5, n0, n1, n2, n3, n4, n5,
                  s_ref, d_ref):
    x = cells_ref[...]
    s_ref[...] = _bd_mlp(x, (c0, c1, c2, c3, c4, c5))
    d_ref[...] = _bd_mlp(x, (n0, n1, n2, n3, n4, n5))


def _full(shape):
    return pl.BlockSpec(shape, lambda i: (0,) * len(shape))


def _percell(cells_pack, p_edc, p_edn):
    wargs = []
    for p in (p_edc, p_edn):
        wargs += list(_kron_mlp(p))
    wspecs = [_full(a.shape) for a in wargs]
    return pl.pallas_call(
        _percell_body,
        grid=(NP // _RB,),
        in_specs=[pl.BlockSpec((_RBP, 128), lambda i: (i, 0))] + wspecs,
        out_specs=[pl.BlockSpec((_RBP, 128), lambda i: (i, 0))] * 2,
        out_shape=[jax.ShapeDtypeStruct((NP // 8, 128), jnp.float32)] * 2,
    )(cells_pack, *wargs)


@functools.lru_cache(maxsize=1)
def _make_gather_k():
    return functools.partial(
        pl.kernel,
        out_type=tuple(jax.ShapeDtypeStruct((CHUNK_E, SD), jnp.float32)
                       for _ in range(4)),
        mesh=_sc_mesh(),
        scratch_types=[
            pltpu.VMEM((GRP // CH, CH), jnp.int32),
            pltpu.VMEM((GRP // CH, CH), jnp.int32),
            pltpu.VMEM((GRP, SD), jnp.float32),
            pltpu.VMEM((GRP, SD), jnp.float32),
            pltpu.VMEM((GRP, SD), jnp.float32),
            pltpu.VMEM((GRP, SD), jnp.float32),
            pltpu.SemaphoreType.DMA,
        ],
        compiler_params=_sc_params,
    )(_gather_body)


def _gather_body(cell_t, edc_t, edn_t, srcg, dstg,
                 ga_out, gx_out, gb_out, gy_out,
                 idx_s, idx_d, buf_a, buf_x, buf_b, buf_y, sem):
    wid = lax.axis_index("c") * NS + lax.axis_index("s")
    rows_per_grp = GRP // CH

    def body(g, carry):
        r0 = wid * (PER_TILE // CH) + g * rows_per_grp
        pltpu.sync_copy(srcg.at[pl.ds(r0, rows_per_grp)], idx_s)
        pltpu.sync_copy(dstg.at[pl.ds(r0, rows_per_grp)], idx_d)
        descs = []
        for j in range(rows_per_grp):
            sl = pl.ds(j * CH, CH)
            descs.append(pltpu.async_copy(
                cell_t.at[idx_s.at[j]], buf_a.at[sl], sem))
            descs.append(pltpu.async_copy(
                edc_t.at[idx_s.at[j]], buf_x.at[sl], sem))
            descs.append(pltpu.async_copy(
                cell_t.at[idx_d.at[j]], buf_b.at[sl], sem))
            descs.append(pltpu.async_copy(
                edn_t.at[idx_d.at[j]], buf_y.at[sl], sem))
        for dsc in descs:
            dsc.wait()
        e0 = wid * PER_TILE + g * GRP
        pltpu.sync_copy(buf_a, ga_out.at[pl.ds(e0, GRP)])
        pltpu.sync_copy(buf_x, gx_out.at[pl.ds(e0, GRP)])
        pltpu.sync_copy(buf_b, gb_out.at[pl.ds(e0, GRP)])
        pltpu.sync_copy(buf_y, gy_out.at[pl.ds(e0, GRP)])
        return carry

    lax.fori_loop(0, GROUPS, body, 0)


_EB = 8192
_EBR = _EB // 8


def _effect_body(ga_ref, gx_ref, gb_ref, gy_ref,
                 bda, bdb, bdd, b0t, w1t, b1t, w2t, b2t, out_ref):
    h = jnp.dot(ga_ref[...], bda[...], preferred_element_type=jnp.float32)
    h += jnp.dot(gb_ref[...], bdb[...], preferred_element_type=jnp.float32)
    h += jnp.dot(gx_ref[...] * gy_ref[...], bdd[...],
                 preferred_element_type=jnp.float32)
    h = jax.nn.relu(h + b0t[...])
    h = jax.nn.relu(jnp.dot(h, w1t[...], preferred_element_type=jnp.float32)
                    + b1t[...])
    out_ref[...] = (jnp.dot(h, w2t[...], preferred_element_type=jnp.float32)
                    + b2t[...])


def _effects(ga8, gx8, gb8, gy8, p_eff):
    w0, b0, w1, b1, w2, b2 = p_eff
    i8 = jnp.eye(8, dtype=jnp.float32)
    bda = jnp.kron(i8, w0[:SD])
    bdb = jnp.kron(i8, w0[SD:2 * SD])
    bdd = jnp.kron(i8, w0[2 * SD:])
    w1t = jnp.kron(i8, w1)
    w2t = jnp.kron(i8, w2)
    b0t = jnp.tile(b0, 8)[None, :]
    b1t = jnp.tile(b1, 8)[None, :]
    b2t = jnp.tile(b2, 8)[None, :]
    wargs = [bda, bdb, bdd, b0t, w1t, b1t, w2t, b2t]
    wspecs = [_full(a.shape) for a in wargs]
    return pl.pallas_call(
        _effect_body,
        grid=(CHUNK_E // _EB,),
        in_specs=[pl.BlockSpec((_EBR, 128), lambda i: (i, 0))] * 4 + wspecs,
        out_specs=pl.BlockSpec((_EBR, 128), lambda i: (i, 0)),
        out_shape=jax.ShapeDtypeStruct((CHUNK_E // 8, 128), jnp.float32),
    )(ga8, gx8, gb8, gy8, *wargs)


@functools.lru_cache(maxsize=1)
def _make_segsum_k():
    return functools.partial(
        pl.kernel,
        out_type=jax.ShapeDtypeStruct((NC, NT, ED), jnp.float32),
        mesh=_sc_mesh(),
        scratch_types=[
            pltpu.VMEM((GRP // CH, CH), jnp.int32),
            pltpu.VMEM((GRP, ED), jnp.float32),
            pltpu.VMEM_SHARED((NT, ED), jnp.float32),
            pltpu.SemaphoreType.DMA,
        ],
        compiler_params=_sc_params,
    )(_segsum_body)


def _segsum_body(eff_hbm, srcn, out_hbm, idx_v, ebuf, tot_sh, sem):
    c = lax.axis_index("c")
    s = lax.axis_index("s")
    rows_per_grp = GRP // CH

    def zb(i, carry):
        ebuf[i] = jnp.zeros((ED,), jnp.float32)
        return carry

    lax.fori_loop(0, GRP, zb, 0)
    for k in range(ZR // GRP):
        pltpu.sync_copy(ebuf, tot_sh.at[pl.ds(s * ZR + k * GRP, GRP)])
    pltpu.sync_copy(ebuf.at[pl.ds(0, ZR % GRP)],
                    tot_sh.at[pl.ds(s * ZR + (ZR // GRP) * GRP, ZR % GRP)])
    plsc.subcore_barrier()

    wid = c * NS + s

    def body(g, carry):
        r0 = wid * (PER_TILE // CH) + g * rows_per_grp
        pltpu.sync_copy(srcn.at[pl.ds(r0, rows_per_grp)], idx_v)
        e0 = wid * PER_TILE + g * GRP
        pltpu.sync_copy(eff_hbm.at[pl.ds(e0, GRP)], ebuf)
        for j in range(rows_per_grp):
            pltpu.sync_copy(ebuf.at[pl.ds(j * CH, CH)],
                            tot_sh.at[idx_v.at[j]], add=True)
        return carry

    lax.fori_loop(0, GROUPS, body, 0)
    plsc.subcore_barrier()

    pltpu.sync_copy(tot_sh.at[pl.ds(s * ZR, ZR)],
                    out_hbm.at[c].at[pl.ds(s * ZR, ZR)])


def _apply_body(cells_ref, tota_ref, totb_ref,
                a0, a1, a2, a3, a4, a5, e0, e1, e2, e3, e4, e5,
                bc, bt, bd, q0, q1, q2, q3, q4, out_ref):
    x = cells_ref[...]
    tot = (tota_ref[0] + tota_ref[1]) + (totb_ref[0] + totb_ref[1])
    adotp = (_bd_mlp(x, (a0, a1, a2, a3, a4, a5))
             * _bd_mlp(tot, (e0, e1, e2, e3, e4, e5)))
    h = jnp.dot(x, bc[...], preferred_element_type=jnp.float32)
    h += jnp.dot(tot, bt[...], preferred_element_type=jnp.float32)
    h += jnp.dot(adotp, bd[...], preferred_element_type=jnp.float32)
    h = jax.nn.relu(h + q0[...])
    h = jax.nn.relu(jnp.dot(h, q1[...], preferred_element_type=jnp.float32)
                    + q2[...])
    out_ref[...] = (jnp.dot(h, q3[...], preferred_element_type=jnp.float32)
                    + q4[...])


def _apply(cells_pack, tot8a, tot8b, p_adc, p_ade, p_app):
    w0, b0, w1, b1, w2, b2 = p_app
    i8 = jnp.eye(8, dtype=jnp.float32)
    bc = jnp.kron(i8, w0[:SD])
    bt = jnp.kron(i8, w0[SD:SD + ED])
    bd = jnp.kron(i8, w0[SD + ED:])
    q0 = jnp.tile(b0, 8)[None, :]
    q1 = jnp.kron(i8, w1)
    q2 = jnp.tile(b1, 8)[None, :]
    q3 = jnp.kron(i8, w2)
    q4 = jnp.tile(b2, 8)[None, :]
    wargs = (list(_kron_mlp(p_adc)) + list(_kron_mlp(p_ade))
             + [bc, bt, bd, q0, q1, q2, q3, q4])
    wspecs = [_full(a.shape) for a in wargs]
    return pl.pallas_call(
        _apply_body,
        grid=(NP // _RB,),
        in_specs=[pl.BlockSpec((_RBP, 128), lambda i: (i, 0)),
                  pl.BlockSpec((NC, _RBP, 128), lambda i: (0, i, 0)),
                  pl.BlockSpec((NC, _RBP, 128), lambda i: (0, i, 0))] + wspecs,
        out_specs=pl.BlockSpec((_RBP, 128), lambda i: (i, 0)),
        out_shape=jax.ShapeDtypeStruct((NP // 8, 128), jnp.float32),
    )(cells_pack, tot8a, tot8b, *wargs)


def kernel(grid_obs, start_hidden, p_edc, p_edn, p_eff, p_adc, p_ade, p_app,
           effect_src, effect_dst):
    cells = jnp.concatenate([grid_obs[0], start_hidden], axis=-1)
    cells_pack = jnp.pad(cells, ((0, NP - N), (0, 0))).reshape(NP // 8, 128)

    src32 = effect_src.astype(jnp.int32)
    dst32 = effect_dst.astype(jnp.int32)
    pad = EP - E
    src_gf = jnp.concatenate([src32, jnp.zeros((pad,), jnp.int32)])
    dst_gf = jnp.concatenate([dst32, jnp.zeros((pad,), jnp.int32)])
    src_nf = jnp.concatenate([src32, jnp.full((pad,), N, jnp.int32)])
    rows_c = CHUNK_E // CH

    edc8, edn8 = _percell(cells_pack, p_edc, p_edn)
    cell_t = cells_pack.reshape(NP, SD)
    edc_t = edc8.reshape(NP, SD)
    edn_t = edn8.reshape(NP, SD)

    tots = []
    effs = []
    for k in range(SPLIT):
        sl = slice(k * CHUNK_E, (k + 1) * CHUNK_E)
        src_g = src_gf[sl].reshape(rows_c, CH)
        dst_g = dst_gf[sl].reshape(rows_c, CH)
        src_n = src_nf[sl].reshape(rows_c, CH)
        ga, gx, gb, gy = _make_gather_k()(cell_t, edc_t, edn_t, src_g, dst_g)
        eff8 = _effects(ga.reshape(CHUNK_E // 8, 128),
                        gx.reshape(CHUNK_E // 8, 128),
                        gb.reshape(CHUNK_E // 8, 128),
                        gy.reshape(CHUNK_E // 8, 128), p_eff)
        tots.append(_make_segsum_k()(eff8.reshape(CHUNK_E, ED), src_n))

    out_pack = _apply(cells_pack,
                      tots[0].reshape(NC, NT // 8, 128),
                      tots[1].reshape(NC, NT // 8, 128),
                      p_adc, p_ade, p_app)
    return out_pack.reshape(NP, SD)[:N, :OBS].reshape(1, 1, N, OBS)


# speedup vs baseline: 1.3258x; 1.3258x over previous
import functools

import jax
import jax.numpy as jnp
from jax import lax
from jax.experimental import pallas as pl
from jax.experimental.pallas import tpu as pltpu
from jax.experimental.pallas import tpu_sc as plsc

N = 100000
E = 1600000
OBS = 8
SD = 16
ED = 16
DP = 16

NC = 2
NS = 16
NW = NC * NS
CH = 128
GRP = 1024
GROUPS = 50
SPLIT = 1
PER_TILE = GROUPS * GRP
CHUNK_E = NW * PER_TILE
EP = SPLIT * CHUNK_E
NP = 102400
NT = 102400
ZR = NT // NS

_sc_params = pltpu.CompilerParams(use_tc_tiling_on_sc=False)


@functools.lru_cache(maxsize=1)
def _sc_mesh():
    return plsc.VectorSubcoreMesh(core_axis_name="c", subcore_axis_name="s",
                                  num_cores=NC, num_subcores=NS)


_RB = 2048
_RBP = _RB // 8


def _kron_mlp(params):
    w0, b0, w1, b1, w2, b2 = params
    i8 = jnp.eye(8, dtype=jnp.float32)
    return (jnp.kron(i8, w0), jnp.tile(b0, 8)[None, :],
            jnp.kron(i8, w1), jnp.tile(b1, 8)[None, :],
            jnp.kron(i8, w2), jnp.tile(b2, 8)[None, :])


def _bd_mlp(x, kw):
    bd0, b0t, bd1, b1t, bd2, b2t = kw
    h = jax.nn.relu(jnp.dot(x, bd0[...], preferred_element_type=jnp.float32)
                    + b0t[...])
    h = jax.nn.relu(jnp.dot(h, bd1[...], preferred_element_type=jnp.float32)
                    + b1t[...])
    return jnp.dot(h, bd2[...], preferred_element_type=jnp.float32) + b2t[...]


def _pack_bf16(hi_f32, lo_f32):
    hi = lax.bitcast_convert_type(hi_f32.astype(jnp.bfloat16), jnp.uint16)
    lo = lax.bitcast_convert_type(lo_f32.astype(jnp.bfloat16), jnp.uint16)
    word = (hi.astype(jnp.uint32) << 16) | lo.astype(jnp.uint32)
    return lax.bitcast_convert_type(word, jnp.int32)


def _unpack_hi(word_i32):
    w = lax.bitcast_convert_type(word_i32, jnp.uint32)
    return lax.bitcast_convert_type(w & jnp.uint32(0xFFFF0000), jnp.float32)


def _unpack_lo(word_i32):
    w = lax.bitcast_convert_type(word_i32, jnp.uint32)
    return lax.bitcast_convert_type(w << 16, jnp.float32)


def _percell_body(cells_ref,
                  c0, c1, c2, c3, c4, c5, n0, n1, n2, n3, n4, n5,
                  s_ref, d_ref):
    x = cells_ref[...]
    s_ref[...] = _pack_bf16(x, _bd_mlp(x, (c0, c1, c2, c3, c4, c5)))
    d_ref[...] = _pack_bf16(x, _bd_mlp(x, (n0, n1, n2, n3, n4, n5)))


def _full(shape):
    return pl.BlockSpec(shape, lambda i: (0,) * len(shape))


def _percell(cells_pack, p_edc, p_edn):
    wargs = []
    for p in (p_edc, p_edn):
        wargs += list(_kron_mlp(p))
    wspecs = [_full(a.shape) for a in wargs]
    return pl.pallas_call(
        _percell_body,
        grid=(NP // _RB,),
        in_specs=[pl.BlockSpec((_RBP, 128), lambda i: (i, 0))] + wspecs,
        out_specs=[pl.BlockSpec((_RBP, 128), lambda i: (i, 0))] * 2,
        out_shape=[jax.ShapeDtypeStruct((NP // 8, 128), jnp.int32)] * 2,
    )(cells_pack, *wargs)


@functools.lru_cache(maxsize=1)
def _make_gather_k():
    return functools.partial(
        pl.kernel,
        out_type=tuple(jax.ShapeDtypeStruct((CHUNK_E, SD), jnp.int32)
                       for _ in range(2)),
        mesh=_sc_mesh(),
        scratch_types=[
            pltpu.VMEM((GRP // CH, CH), jnp.int32),
            pltpu.VMEM((GRP // CH, CH), jnp.int32),
            pltpu.VMEM((GRP, SD), jnp.int32),
            pltpu.VMEM((GRP, SD), jnp.int32),
            pltpu.SemaphoreType.DMA,
        ],
        compiler_params=_sc_params,
    )(_gather_body)


def _gather_body(sx_t, dy_t, srcg, dstg, gsx_out, gdy_out,
                 idx_s, idx_d, buf_s, buf_d, sem):
    wid = lax.axis_index("c") * NS + lax.axis_index("s")
    rows_per_grp = GRP // CH

    def body(g, carry):
        r0 = wid * (PER_TILE // CH) + g * rows_per_grp
        pltpu.sync_copy(srcg.at[pl.ds(r0, rows_per_grp)], idx_s)
        pltpu.sync_copy(dstg.at[pl.ds(r0, rows_per_grp)], idx_d)
        descs = []
        for j in range(rows_per_grp):
            sl = pl.ds(j * CH, CH)
            descs.append(pltpu.async_copy(
                sx_t.at[idx_s.at[j]], buf_s.at[sl], sem))
            descs.append(pltpu.async_copy(
                dy_t.at[idx_d.at[j]], buf_d.at[sl], sem))
        for dsc in descs:
            dsc.wait()
        e0 = wid * PER_TILE + g * GRP
        pltpu.sync_copy(buf_s, gsx_out.at[pl.ds(e0, GRP)])
        pltpu.sync_copy(buf_d, gdy_out.at[pl.ds(e0, GRP)])
        return carry

    lax.fori_loop(0, GROUPS, body, 0)


_EB = 8192
_EBR = _EB // 8


def _effect_body(gsx_ref, gdy_ref,
                 bda, bdb, bdd, b0t, w1t, b1t, w2t, b2t, out_ref):
    sx = gsx_ref[...]
    dy = gdy_ref[...]
    h = jnp.dot(_unpack_hi(sx), bda[...], preferred_element_type=jnp.float32)
    h += jnp.dot(_unpack_hi(dy), bdb[...], preferred_element_type=jnp.float32)
    h += jnp.dot(_unpack_lo(sx) * _unpack_lo(dy), bdd[...],
                 preferred_element_type=jnp.float32)
    h = jax.nn.relu(h + b0t[...])
    h = jax.nn.relu(jnp.dot(h, w1t[...], preferred_element_type=jnp.float32)
                    + b1t[...])
    out_ref[...] = (jnp.dot(h, w2t[...], preferred_element_type=jnp.float32)
                    + b2t[...])


def _effects(gsx8, gdy8, p_eff):
    w0, b0, w1, b1, w2, b2 = p_eff
    i8 = jnp.eye(8, dtype=jnp.float32)
    bda = jnp.kron(i8, w0[:SD])
    bdb = jnp.kron(i8, w0[SD:2 * SD])
    bdd = jnp.kron(i8, w0[2 * SD:])
    w1t = jnp.kron(i8, w1)
    w2t = jnp.kron(i8, w2)
    b0t = jnp.tile(b0, 8)[None, :]
    b1t = jnp.tile(b1, 8)[None, :]
    b2t = jnp.tile(b2, 8)[None, :]
    wargs = [bda, bdb, bdd, b0t, w1t, b1t, w2t, b2t]
    wspecs = [_full(a.shape) for a in wargs]
    return pl.pallas_call(
        _effect_body,
        grid=(CHUNK_E // _EB,),
        in_specs=[pl.BlockSpec((_EBR, 128), lambda i: (i, 0))] * 2 + wspecs,
        out_specs=pl.BlockSpec((_EBR, 128), lambda i: (i, 0)),
        out_shape=jax.ShapeDtypeStruct((CHUNK_E // 8, 128), jnp.float32),
    )(gsx8, gdy8, *wargs)


@functools.lru_cache(maxsize=1)
def _make_segsum_k():
    return functools.partial(
        pl.kernel,
        out_type=jax.ShapeDtypeStruct((NC, NT, ED), jnp.float32),
        mesh=_sc_mesh(),
        scratch_types=[
            pltpu.VMEM((GRP // CH, CH), jnp.int32),
            pltpu.VMEM((GRP, ED), jnp.float32),
            pltpu.VMEM_SHARED((NT, ED), jnp.float32),
            pltpu.SemaphoreType.DMA,
        ],
        compiler_params=_sc_params,
    )(_segsum_body)


def _segsum_body(eff_hbm, srcn, out_hbm, idx_v, ebuf, tot_sh, sem):
    c = lax.axis_index("c")
    s = lax.axis_index("s")
    rows_per_grp = GRP // CH

    def zb(i, carry):
        ebuf[i] = jnp.zeros((ED,), jnp.float32)
        return carry

    lax.fori_loop(0, GRP, zb, 0)
    for k in range(ZR // GRP):
        pltpu.sync_copy(ebuf, tot_sh.at[pl.ds(s * ZR + k * GRP, GRP)])
    pltpu.sync_copy(ebuf.at[pl.ds(0, ZR % GRP)],
                    tot_sh.at[pl.ds(s * ZR + (ZR // GRP) * GRP, ZR % GRP)])
    plsc.subcore_barrier()

    wid = c * NS + s

    def body(g, carry):
        r0 = wid * (PER_TILE // CH) + g * rows_per_grp
        pltpu.sync_copy(srcn.at[pl.ds(r0, rows_per_grp)], idx_v)
        e0 = wid * PER_TILE + g * GRP
        pltpu.sync_copy(eff_hbm.at[pl.ds(e0, GRP)], ebuf)
        for j in range(rows_per_grp):
            pltpu.sync_copy(ebuf.at[pl.ds(j * CH, CH)],
                            tot_sh.at[idx_v.at[j]], add=True)
        return carry

    lax.fori_loop(0, GROUPS, body, 0)
    plsc.subcore_barrier()

    pltpu.sync_copy(tot_sh.at[pl.ds(s * ZR, ZR)],
                    out_hbm.at[c].at[pl.ds(s * ZR, ZR)])


def _apply_body(cells_ref, tot_ref,
                a0, a1, a2, a3, a4, a5, e0, e1, e2, e3, e4, e5,
                bc, bt, bd, q0, q1, q2, q3, q4, out_ref):
    x = cells_ref[...]
    tot = tot_ref[0] + tot_ref[1]
    adotp = (_bd_mlp(x, (a0, a1, a2, a3, a4, a5))
             * _bd_mlp(tot, (e0, e1, e2, e3, e4, e5)))
    h = jnp.dot(x, bc[...], preferred_element_type=jnp.float32)
    h += jnp.dot(tot, bt[...], preferred_element_type=jnp.float32)
    h += jnp.dot(adotp, bd[...], preferred_element_type=jnp.float32)
    h = jax.nn.relu(h + q0[...])
    h = jax.nn.relu(jnp.dot(h, q1[...], preferred_element_type=jnp.float32)
                    + q2[...])
    out_ref[...] = (jnp.dot(h, q3[...], preferred_element_type=jnp.float32)
                    + q4[...])


def _apply(cells_pack, tot8, p_adc, p_ade, p_app):
    w0, b0, w1, b1, w2, b2 = p_app
    i8 = jnp.eye(8, dtype=jnp.float32)
    bc = jnp.kron(i8, w0[:SD])
    bt = jnp.kron(i8, w0[SD:SD + ED])
    bd = jnp.kron(i8, w0[SD + ED:])
    q0 = jnp.tile(b0, 8)[None, :]
    q1 = jnp.kron(i8, w1)
    q2 = jnp.tile(b1, 8)[None, :]
    q3 = jnp.kron(i8, w2)
    q4 = jnp.tile(b2, 8)[None, :]
    wargs = (list(_kron_mlp(p_adc)) + list(_kron_mlp(p_ade))
             + [bc, bt, bd, q0, q1, q2, q3, q4])
    wspecs = [_full(a.shape) for a in wargs]
    return pl.pallas_call(
        _apply_body,
        grid=(NP // _RB,),
        in_specs=[pl.BlockSpec((_RBP, 128), lambda i: (i, 0)),
                  pl.BlockSpec((NC, _RBP, 128), lambda i: (0, i, 0))] + wspecs,
        out_specs=pl.BlockSpec((_RBP, 128), lambda i: (i, 0)),
        out_shape=jax.ShapeDtypeStruct((NP // 8, 128), jnp.float32),
    )(cells_pack, tot8, *wargs)


def kernel(grid_obs, start_hidden, p_edc, p_edn, p_eff, p_adc, p_ade, p_app,
           effect_src, effect_dst):
    cells = jnp.concatenate([grid_obs[0], start_hidden], axis=-1)
    cells_pack = jnp.pad(cells, ((0, NP - N), (0, 0))).reshape(NP // 8, 128)

    src32 = effect_src.astype(jnp.int32)
    dst32 = effect_dst.astype(jnp.int32)
    pad = EP - E
    src_gf = jnp.concatenate([src32, jnp.zeros((pad,), jnp.int32)])
    dst_gf = jnp.concatenate([dst32, jnp.zeros((pad,), jnp.int32)])
    src_nf = jnp.concatenate([src32, jnp.full((pad,), N, jnp.int32)])
    rows_c = CHUNK_E // CH

    sx8, dy8 = _percell(cells_pack, p_edc, p_edn)
    sx_t = sx8.reshape(NP, SD)
    dy_t = dy8.reshape(NP, SD)

    tots = []
    for k in range(SPLIT):
        sl = slice(k * CHUNK_E, (k + 1) * CHUNK_E)
        src_g = src_gf[sl].reshape(rows_c, CH)
        dst_g = dst_gf[sl].reshape(rows_c, CH)
        src_n = src_nf[sl].reshape(rows_c, CH)
        gsx, gdy = _make_gather_k()(sx_t, dy_t, src_g, dst_g)
        eff8 = _effects(gsx.reshape(CHUNK_E // 8, 128),
                        gdy.reshape(CHUNK_E // 8, 128), p_eff)
        tots.append(_make_segsum_k()(eff8.reshape(CHUNK_E, ED), src_n))

    tot8 = tots[0] if SPLIT == 1 else None
    out_pack = _apply(cells_pack, tot8.reshape(NC, NT // 8, 128),
                      p_adc, p_ade, p_app)
    return out_pack.reshape(NP, SD)[:N, :OBS].reshape(1, 1, N, OBS)


# speedup vs baseline: 1.4956x; 1.1281x over previous
import functools

import jax
import jax.numpy as jnp
from jax import lax
from jax.experimental import pallas as pl
from jax.experimental.pallas import tpu as pltpu
from jax.experimental.pallas import tpu_sc as plsc

N = 100000
E = 1600000
OBS = 8
SD = 16
ED = 16
DP = 16

NC = 2
NS = 16
NW = NC * NS
CH = 128
GRP = 1024
GROUPS = 50
SPLIT = 1
PER_TILE = GROUPS * GRP
CHUNK_E = NW * PER_TILE
EP = SPLIT * CHUNK_E
NP = 102400
NT = 102400
ZR = NT // NS

_sc_params = pltpu.CompilerParams(use_tc_tiling_on_sc=False)


@functools.lru_cache(maxsize=1)
def _sc_mesh():
    return plsc.VectorSubcoreMesh(core_axis_name="c", subcore_axis_name="s",
                                  num_cores=NC, num_subcores=NS)


_RB = 2048
_RBP = _RB // 8


def _kron_mlp(params):
    w0, b0, w1, b1, w2, b2 = params
    i8 = jnp.eye(8, dtype=jnp.float32)
    return (jnp.kron(i8, w0), jnp.tile(b0, 8)[None, :],
            jnp.kron(i8, w1), jnp.tile(b1, 8)[None, :],
            jnp.kron(i8, w2), jnp.tile(b2, 8)[None, :])


def _bd_mlp(x, kw):
    bd0, b0t, bd1, b1t, bd2, b2t = kw
    h = jax.nn.relu(jnp.dot(x, bd0[...], preferred_element_type=jnp.float32)
                    + b0t[...])
    h = jax.nn.relu(jnp.dot(h, bd1[...], preferred_element_type=jnp.float32)
                    + b1t[...])
    return jnp.dot(h, bd2[...], preferred_element_type=jnp.float32) + b2t[...]


def _pack_bf16(hi_f32, lo_f32):
    hi = lax.bitcast_convert_type(hi_f32.astype(jnp.bfloat16), jnp.uint16)
    lo = lax.bitcast_convert_type(lo_f32.astype(jnp.bfloat16), jnp.uint16)
    word = (hi.astype(jnp.uint32) << 16) | lo.astype(jnp.uint32)
    return lax.bitcast_convert_type(word, jnp.int32)


def _unpack_hi(word_i32):
    w = lax.bitcast_convert_type(word_i32, jnp.uint32)
    return lax.bitcast_convert_type(w & jnp.uint32(0xFFFF0000), jnp.float32)


def _unpack_lo(word_i32):
    w = lax.bitcast_convert_type(word_i32, jnp.uint32)
    return lax.bitcast_convert_type(w << 16, jnp.float32)


def _percell_body(cells_ref,
                  c0, c1, c2, c3, c4, c5, n0, n1, n2, n3, n4, n5,
                  s_ref, d_ref):
    x = cells_ref[...]
    s_ref[...] = _pack_bf16(x, _bd_mlp(x, (c0, c1, c2, c3, c4, c5)))
    d_ref[...] = _pack_bf16(x, _bd_mlp(x, (n0, n1, n2, n3, n4, n5)))


def _full(shape):
    return pl.BlockSpec(shape, lambda i: (0,) * len(shape))


def _percell(cells_pack, p_edc, p_edn):
    wargs = []
    for p in (p_edc, p_edn):
        wargs += list(_kron_mlp(p))
    wspecs = [_full(a.shape) for a in wargs]
    return pl.pallas_call(
        _percell_body,
        grid=(NP // _RB,),
        in_specs=[pl.BlockSpec((_RBP, 128), lambda i: (i, 0))] + wspecs,
        out_specs=[pl.BlockSpec((_RBP, 128), lambda i: (i, 0))] * 2,
        out_shape=[jax.ShapeDtypeStruct((NP // 8, 128), jnp.int32)] * 2,
    )(cells_pack, *wargs)


@functools.lru_cache(maxsize=1)
def _make_gather_k():
    return functools.partial(
        pl.kernel,
        out_type=tuple(jax.ShapeDtypeStruct((CHUNK_E, SD), jnp.int32)
                       for _ in range(2)),
        mesh=_sc_mesh(),
        scratch_types=[
            pltpu.VMEM((GRP // CH, CH), jnp.int32),
            pltpu.VMEM((GRP // CH, CH), jnp.int32),
            pltpu.VMEM((GRP, SD), jnp.int32),
            pltpu.VMEM((GRP, SD), jnp.int32),
            pltpu.SemaphoreType.DMA,
        ],
        compiler_params=_sc_params,
    )(_gather_body)


def _gather_body(sx_t, dy_t, srcg, dstg, gsx_out, gdy_out,
                 idx_s, idx_d, buf_s, buf_d, sem):
    wid = lax.axis_index("c") * NS + lax.axis_index("s")
    rows_per_grp = GRP // CH

    def body(g, carry):
        r0 = wid * (PER_TILE // CH) + g * rows_per_grp
        pltpu.sync_copy(srcg.at[pl.ds(r0, rows_per_grp)], idx_s)
        pltpu.sync_copy(dstg.at[pl.ds(r0, rows_per_grp)], idx_d)
        descs = []
        for j in range(rows_per_grp):
            sl = pl.ds(j * CH, CH)
            descs.append(pltpu.async_copy(
                sx_t.at[idx_s.at[j]], buf_s.at[sl], sem))
            descs.append(pltpu.async_copy(
                dy_t.at[idx_d.at[j]], buf_d.at[sl], sem))
        for dsc in descs:
            dsc.wait()
        e0 = wid * PER_TILE + g * GRP
        pltpu.sync_copy(buf_s, gsx_out.at[pl.ds(e0, GRP)])
        pltpu.sync_copy(buf_d, gdy_out.at[pl.ds(e0, GRP)])
        return carry

    lax.fori_loop(0, GROUPS, body, 0)


_EB = 8192
_EBR = _EB // 8


def _effect_body(gsx_ref, gdy_ref,
                 bda, bdb, bdd, b0t, w1t, b1t, w2t, b2t, out_ref):
    sx = gsx_ref[...]
    dy = gdy_ref[...]
    h = jnp.dot(_unpack_hi(sx), bda[...], preferred_element_type=jnp.float32)
    h += jnp.dot(_unpack_hi(dy), bdb[...], preferred_element_type=jnp.float32)
    h += jnp.dot(_unpack_lo(sx) * _unpack_lo(dy), bdd[...],
                 preferred_element_type=jnp.float32)
    h = jax.nn.relu(h + b0t[...])
    h = jax.nn.relu(jnp.dot(h, w1t[...], preferred_element_type=jnp.float32)
                    + b1t[...])
    out_ref[...] = (jnp.dot(h, w2t[...], preferred_element_type=jnp.float32)
                    + b2t[...])


def _effects(gsx8, gdy8, p_eff):
    w0, b0, w1, b1, w2, b2 = p_eff
    i8 = jnp.eye(8, dtype=jnp.float32)
    bda = jnp.kron(i8, w0[:SD])
    bdb = jnp.kron(i8, w0[SD:2 * SD])
    bdd = jnp.kron(i8, w0[2 * SD:])
    w1t = jnp.kron(i8, w1)
    w2t = jnp.kron(i8, w2)
    b0t = jnp.tile(b0, 8)[None, :]
    b1t = jnp.tile(b1, 8)[None, :]
    b2t = jnp.tile(b2, 8)[None, :]
    wargs = [bda, bdb, bdd, b0t, w1t, b1t, w2t, b2t]
    wspecs = [_full(a.shape) for a in wargs]
    return pl.pallas_call(
        _effect_body,
        grid=(CHUNK_E // _EB,),
        in_specs=[pl.BlockSpec((_EBR, 128), lambda i: (i, 0))] * 2 + wspecs,
        out_specs=pl.BlockSpec((_EBR, 128), lambda i: (i, 0)),
        out_shape=jax.ShapeDtypeStruct((CHUNK_E // 8, 128), jnp.float32),
    )(gsx8, gdy8, *wargs)


@functools.lru_cache(maxsize=1)
def _make_segsum_k():
    return functools.partial(
        pl.kernel,
        out_type=jax.ShapeDtypeStruct((NC, NT, ED), jnp.float32),
        mesh=_sc_mesh(),
        scratch_types=[
            pltpu.VMEM((GRP // CH, CH), jnp.int32),
            pltpu.VMEM((GRP, ED), jnp.float32),
            pltpu.VMEM_SHARED((NT, ED), jnp.float32),
            pltpu.SemaphoreType.DMA,
        ],
        compiler_params=_sc_params,
    )(_segsum_body)


def _segsum_body(eff_hbm, srcn, out_hbm, idx_v, ebuf, tot_sh, sem):
    c = lax.axis_index("c")
    s = lax.axis_index("s")
    rows_per_grp = GRP // CH

    def zb(i, carry):
        ebuf[i] = jnp.zeros((ED,), jnp.float32)
        return carry

    lax.fori_loop(0, GRP, zb, 0)
    for k in range(ZR // GRP):
        pltpu.sync_copy(ebuf, tot_sh.at[pl.ds(s * ZR + k * GRP, GRP)])
    pltpu.sync_copy(ebuf.at[pl.ds(0, ZR % GRP)],
                    tot_sh.at[pl.ds(s * ZR + (ZR // GRP) * GRP, ZR % GRP)])
    plsc.subcore_barrier()

    wid = c * NS + s

    def body(g, carry):
        r0 = wid * (PER_TILE // CH) + g * rows_per_grp
        pltpu.sync_copy(srcn.at[pl.ds(r0, rows_per_grp)], idx_v)
        e0 = wid * PER_TILE + g * GRP
        pltpu.sync_copy(eff_hbm.at[pl.ds(e0, GRP)], ebuf)
        for j in range(rows_per_grp):
            pltpu.sync_copy(ebuf.at[pl.ds(j * CH, CH)],
                            tot_sh.at[idx_v.at[j]], add=True)
        return carry

    lax.fori_loop(0, GROUPS, body, 0)
    plsc.subcore_barrier()

    pltpu.sync_copy(tot_sh.at[pl.ds(s * ZR, ZR)],
                    out_hbm.at[c].at[pl.ds(s * ZR, ZR)])


def _apply_body(cells_ref, tot_ref,
                a0, a1, a2, a3, a4, a5, e0, e1, e2, e3, e4, e5,
                bc, bt, bd, q0, q1, q2, q3, q4, out_ref):
    x = cells_ref[...]
    tot = tot_ref[0] + tot_ref[1]
    adotp = (_bd_mlp(x, (a0, a1, a2, a3, a4, a5))
             * _bd_mlp(tot, (e0, e1, e2, e3, e4, e5)))
    h = jnp.dot(x, bc[...], preferred_element_type=jnp.float32)
    h += jnp.dot(tot, bt[...], preferred_element_type=jnp.float32)
    h += jnp.dot(adotp, bd[...], preferred_element_type=jnp.float32)
    h = jax.nn.relu(h + q0[...])
    h = jax.nn.relu(jnp.dot(h, q1[...], preferred_element_type=jnp.float32)
                    + q2[...])
    out_ref[...] = (jnp.dot(h, q3[...], preferred_element_type=jnp.float32)
                    + q4[...])


def _apply(cells_pack, tot8, p_adc, p_ade, p_app):
    w0, b0, w1, b1, w2, b2 = p_app
    i8 = jnp.eye(8, dtype=jnp.float32)
    bc = jnp.kron(i8, w0[:SD])
    bt = jnp.kron(i8, w0[SD:SD + ED])
    bd = jnp.kron(i8, w0[SD + ED:])
    q0 = jnp.tile(b0, 8)[None, :]
    q1 = jnp.kron(i8, w1)
    q2 = jnp.tile(b1, 8)[None, :]
    q3 = jnp.kron(i8, w2)
    q4 = jnp.tile(b2, 8)[None, :]
    wargs = (list(_kron_mlp(p_adc)) + list(_kron_mlp(p_ade))
             + [bc, bt, bd, q0, q1, q2, q3, q4])
    wspecs = [_full(a.shape) for a in wargs]
    return pl.pallas_call(
        _apply_body,
        grid=(NP // _RB,),
        in_specs=[pl.BlockSpec((_RBP, 128), lambda i: (i, 0)),
                  pl.BlockSpec((NC, _RBP, 128), lambda i: (0, i, 0))] + wspecs,
        out_specs=pl.BlockSpec((_RBP, 128), lambda i: (i, 0)),
        out_shape=jax.ShapeDtypeStruct((NP // 8, 128), jnp.float32),
    )(cells_pack, tot8, *wargs)


def kernel(grid_obs, start_hidden, p_edc, p_edn, p_eff, p_adc, p_ade, p_app,
           effect_src, effect_dst):
    cells = jnp.concatenate([grid_obs[0], start_hidden], axis=-1)
    cells_pack = jnp.pad(cells, ((0, NP - N), (0, 0))).reshape(NP // 8, 128)

    src32 = effect_src.astype(jnp.int32)
    dst32 = effect_dst.astype(jnp.int32)
    pad = EP - E
    spread = (jnp.arange(pad, dtype=jnp.int32) * 127) % N
    src_gf = jnp.concatenate([src32, spread])
    dst_gf = jnp.concatenate([dst32, spread])
    src_nf = jnp.concatenate([src32, jnp.full((pad,), N, jnp.int32)])
    rows_c = CHUNK_E // CH

    sx8, dy8 = _percell(cells_pack, p_edc, p_edn)
    sx_t = sx8.reshape(NP, SD)
    dy_t = dy8.reshape(NP, SD)

    tots = []
    for k in range(SPLIT):
        sl = slice(k * CHUNK_E, (k + 1) * CHUNK_E)
        src_g = src_gf[sl].reshape(rows_c, CH)
        dst_g = dst_gf[sl].reshape(rows_c, CH)
        src_n = src_nf[sl].reshape(rows_c, CH)
        gsx, gdy = _make_gather_k()(sx_t, dy_t, src_g, dst_g)
        eff8 = _effects(gsx.reshape(CHUNK_E // 8, 128),
                        gdy.reshape(CHUNK_E // 8, 128), p_eff)
        tots.append(_make_segsum_k()(eff8.reshape(CHUNK_E, ED), src_n))

    tot8 = tots[0] if SPLIT == 1 else None
    out_pack = _apply(cells_pack, tot8.reshape(NC, NT // 8, 128),
                      p_adc, p_ade, p_app)
    return out_pack.reshape(NP, SD)[:N, :OBS].reshape(1, 1, N, OBS)
